# baseline (speedup 1.0000x reference)
import jax
import jax.numpy as jnp
from jax import lax
from jax.experimental import pallas as pl

_N, _B, _D = 2048, 4, 128
_NE = 2050
_BN = 512


def _body(inc_ref, x_ref, tab_ref, out_ref):
    counts_t = jnp.sum(inc_ref[...], axis=-1).T
    iota_ne = lax.broadcasted_iota(jnp.int32, (1, _NE), 1)
    tab = tab_ref[...].astype(jnp.bfloat16)
    for b in range(_B):
        lvl = counts_t[:, b:b + 1] + 1
        oh = (lvl == iota_ne).astype(jnp.bfloat16)
        emb = jnp.dot(oh, tab, preferred_element_type=jnp.float32)
        out_ref[:, b, :] = x_ref[:, b, :] + emb


def kernel(x, node_incidences, pos_embedding):
    return pl.pallas_call(
        _body,
        grid=(_N // _BN,),
        in_specs=[
            pl.BlockSpec((_B, _BN, _N), lambda n: (0, n, 0)),
            pl.BlockSpec((_BN, _B, _D), lambda n: (n, 0, 0)),
            pl.BlockSpec((_NE, _D), lambda n: (0, 0)),
        ],
        out_specs=pl.BlockSpec((_BN, _B, _D), lambda n: (n, 0, 0)),
        out_shape=jax.ShapeDtypeStruct((_N, _B, _D), jnp.float32),
    )(node_incidences, x, pos_embedding)


# speedup vs baseline: 1.0764x; 1.0764x over previous
import jax
import jax.numpy as jnp
from jax import lax
from jax.experimental import pallas as pl
from jax.experimental.pallas import tpu as pltpu

_N, _B, _D = 2048, 4, 128
_NE = 2050
_CI = 128
_NC = _N // _CI
_RING = 3


def _body(inc_hbm, x_hbm, tab_hbm, out_hbm,
          bufs, xbufs, obufs, tabv, isems, xsems, osems, tsem):
    cp_t = pltpu.make_async_copy(tab_hbm, tabv, tsem)
    cp_t.start()

    def inc_start(c):
        return pltpu.make_async_copy(
            inc_hbm.at[:, pl.ds(c * _CI, _CI), :], bufs.at[c % _RING],
            isems.at[c % _RING])

    def x_start(c):
        return pltpu.make_async_copy(
            x_hbm.at[pl.ds(c * _CI, _CI)], xbufs.at[c % _RING],
            xsems.at[c % _RING])

    incs = {0: inc_start(0), 1: inc_start(1)}
    xcs = {0: x_start(0), 1: x_start(1)}
    for c in (0, 1):
        incs[c].start()
        xcs[c].start()
    cp_t.wait()
    tab = tabv[...].astype(jnp.bfloat16)
    iota_ne = lax.broadcasted_iota(jnp.int32, (1, _NE), 1)

    ocs = {}
    for c in range(_NC):
        if c + 2 < _NC:
            incs[c + 2] = inc_start(c + 2)
            incs[c + 2].start()
            xcs[c + 2] = x_start(c + 2)
            xcs[c + 2].start()
        incs[c].wait()
        counts_t = jnp.sum(bufs[c % _RING], axis=-1).T
        if c - 2 >= 0:
            ocs[c - 2].wait()
        xcs[c].wait()
        ob = obufs.at[c % 2]
        for b in range(_B):
            lvl = counts_t[:, b:b + 1] + 1
            oh = (lvl == iota_ne).astype(jnp.bfloat16)
            emb = jnp.dot(oh, tab, preferred_element_type=jnp.float32)
            ob[:, b, :] = xbufs[c % _RING][:, b, :] + emb
        ocs[c] = pltpu.make_async_copy(
            ob, out_hbm.at[pl.ds(c * _CI, _CI)], osems.at[c % 2])
        ocs[c].start()
    ocs[_NC - 2].wait()
    ocs[_NC - 1].wait()


def kernel(x, node_incidences, pos_embedding):
    return pl.pallas_call(
        _body,
        in_specs=[
            pl.BlockSpec(memory_space=pl.ANY),
            pl.BlockSpec(memory_space=pl.ANY),
            pl.BlockSpec(memory_space=pl.ANY),
        ],
        out_specs=pl.BlockSpec(memory_space=pl.ANY),
        out_shape=jax.ShapeDtypeStruct((_N, _B, _D), jnp.float32),
        scratch_shapes=[
            pltpu.VMEM((_RING, _B, _CI, _N), jnp.int32),
            pltpu.VMEM((_RING, _CI, _B, _D), jnp.float32),
            pltpu.VMEM((2, _CI, _B, _D), jnp.float32),
            pltpu.VMEM((_NE, _D), jnp.float32),
            pltpu.SemaphoreType.DMA((_RING,)),
            pltpu.SemaphoreType.DMA((_RING,)),
            pltpu.SemaphoreType.DMA((2,)),
            pltpu.SemaphoreType.DMA,
        ],
    )(node_incidences, x, pos_embedding)
